# baseline (speedup 1.0000x reference)
import functools

import jax
import jax.numpy as jnp
from jax import lax
from jax.experimental import pallas as pl
from jax.experimental.pallas import tpu as pltpu
from jax.experimental.pallas import tpu_sc as plsc

_N, _E, _D = 10000, 320000, 128
_TILES = 32
_NPAD = 10240
_RPT = _NPAD // 16
_CH = 64
_NB = 4
_NGRP = 40
_NCH = _NB * _NGRP
_EPT = _NCH * _CH
_EPAD = _EPT * _TILES
_BN = 1024


@functools.cache
def _sc_aggregate(with_deg):
    mesh = plsc.VectorSubcoreMesh(
        core_axis_name="c", subcore_axis_name="s", num_cores=2, num_subcores=16
    )

    def body(h_hbm, src_hbm, dst_hbm, zrow_hbm, zdeg_hbm, ones_hbm, *rest):
        if with_deg:
            (acc_out, deg_out, sidx, didx, rows_v, ones_v, acc_sh, deg_sh,
             gsem, ssem, dsem, isem) = rest
        else:
            (acc_out, sidx, didx, rows_v, ones_v, acc_sh,
             gsem, ssem, dsem, isem) = rest
        c = lax.axis_index("c")
        s = lax.axis_index("s")
        tid = c * 16 + s
        pltpu.sync_copy(ones_hbm, ones_v)
        pltpu.sync_copy(src_hbm.at[tid, 0], sidx.at[0])
        pltpu.sync_copy(dst_hbm.at[tid, 0], didx.at[0])
        pltpu.sync_copy(zrow_hbm, acc_sh.at[pl.ds(s * _RPT, _RPT)])
        if with_deg:
            pltpu.sync_copy(zdeg_hbm, deg_sh.at[pl.ds(s * _RPT, _RPT)])
        plsc.subcore_barrier()

        def group(g, carry):
            p = g % 2
            gg = jnp.minimum(g + 1, _NGRP - 1)
            pf_s = pltpu.async_copy(
                src_hbm.at[tid, gg], sidx.at[1 - p], isem.at[0])
            pf_d = pltpu.async_copy(
                dst_hbm.at[tid, gg], didx.at[1 - p], isem.at[1])
            gathers = []
            for b in range(_NB):
                gathers.append(pltpu.async_copy(
                    h_hbm.at[sidx.at[p, b]], rows_v.at[b], gsem.at[b]))
            scatters = []
            for b in range(_NB):
                gathers[b].wait()
                scatters.append(pltpu.async_copy(
                    rows_v.at[b], acc_sh.at[didx.at[p, b]], ssem.at[b],
                    add=True))
                if with_deg:
                    scatters.append(pltpu.async_copy(
                        ones_v, deg_sh.at[didx.at[p, b]], dsem.at[b],
                        add=True))
            for cp in scatters:
                cp.wait()
            pf_s.wait()
            pf_d.wait()
            return carry

        lax.fori_loop(0, _NGRP, group, 0)
        plsc.subcore_barrier()
        pltpu.sync_copy(acc_sh.at[pl.ds(s * _RPT, _RPT)],
                        acc_out.at[c, pl.ds(s * _RPT, _RPT)])
        if with_deg:
            pltpu.sync_copy(deg_sh.at[pl.ds(s * _RPT, _RPT)],
                            deg_out.at[c, pl.ds(s * _RPT, _RPT)])

    out_type = [jax.ShapeDtypeStruct((2, _NPAD, _D), jnp.float32)]
    scratch = [
        pltpu.VMEM((2, _NB, _CH), jnp.int32),
        pltpu.VMEM((2, _NB, _CH), jnp.int32),
        pltpu.VMEM((_NB, _CH, _D), jnp.float32),
        pltpu.VMEM((_CH,), jnp.float32),
        pltpu.VMEM_SHARED((_NPAD, _D), jnp.float32),
    ]
    if with_deg:
        out_type.append(jax.ShapeDtypeStruct((2, _NPAD), jnp.float32))
        scratch.append(pltpu.VMEM_SHARED((_NPAD,), jnp.float32))
    scratch += [
        pltpu.SemaphoreType.DMA((_NB,)),
        pltpu.SemaphoreType.DMA((_NB,)),
        pltpu.SemaphoreType.DMA((_NB,)),
        pltpu.SemaphoreType.DMA((2,)),
    ]
    return pl.kernel(body, out_type=tuple(out_type), mesh=mesh,
                     scratch_types=scratch)


def _layer_body(relu, h_ref, acc_ref, d0_ref, d1_ref, ws_ref, wn_ref, b_ref,
                o_ref):
    inv = 1.0 / jnp.maximum(d0_ref[...] + d1_ref[...], 1.0)
    hn = (acc_ref[0] + acc_ref[1]) * inv
    out = (jnp.dot(h_ref[...], ws_ref[...], preferred_element_type=jnp.float32)
           + jnp.dot(hn, wn_ref[...], preferred_element_type=jnp.float32)
           + b_ref[...])
    if relu:
        out = jnp.maximum(out, 0.0)
    o_ref[...] = out


def _layer_call(h, acc, d0, d1, ws, wn, b, relu):
    return pl.pallas_call(
        functools.partial(_layer_body, relu),
        grid=(_NPAD // _BN,),
        in_specs=[
            pl.BlockSpec((_BN, _D), lambda i: (i, 0)),
            pl.BlockSpec((2, _BN, _D), lambda i: (0, i, 0)),
            pl.BlockSpec((_BN, 1), lambda i: (i, 0)),
            pl.BlockSpec((_BN, 1), lambda i: (i, 0)),
            pl.BlockSpec((_D, _D), lambda i: (0, 0)),
            pl.BlockSpec((_D, _D), lambda i: (0, 0)),
            pl.BlockSpec((1, _D), lambda i: (0, 0)),
        ],
        out_specs=pl.BlockSpec((_BN, _D), lambda i: (i, 0)),
        out_shape=jax.ShapeDtypeStruct((_NPAD, _D), jnp.float32),
    )(h, acc, d0, d1, ws, wn, b)


def kernel(x, edge_index, W_self1, W_neigh1, b1, W_self2, W_neigh2, b2):
    src = edge_index[0]
    dst = edge_index[1]
    pad_e = _EPAD - _E
    src_p = jnp.concatenate(
        [src, jnp.zeros((pad_e,), jnp.int32)]).reshape(_TILES, _NGRP, _NB, _CH)
    dst_p = jnp.concatenate(
        [dst, jnp.full((pad_e,), _N, jnp.int32)]).reshape(_TILES, _NGRP, _NB, _CH)
    x_p = jnp.pad(x, ((0, _NPAD - _N), (0, 0)))
    zrow = jnp.zeros((_RPT, _D), jnp.float32)
    zdeg = jnp.zeros((_RPT,), jnp.float32)
    ones = jnp.ones((_CH,), jnp.float32)

    acc1, deg = _sc_aggregate(True)(x_p, src_p, dst_p, zrow, zdeg, ones)
    d0 = deg[0].reshape(_NPAD, 1)
    d1 = deg[1].reshape(_NPAD, 1)
    h1 = _layer_call(x_p, acc1, d0, d1, W_self1, W_neigh1,
                     b1.reshape(1, _D), relu=True)
    (acc2,) = _sc_aggregate(False)(h1, src_p, dst_p, zrow, zdeg, ones)
    h2 = _layer_call(h1, acc2, d0, d1, W_self2, W_neigh2,
                     b2.reshape(1, _D), relu=False)

    fl = (_N * (4 * _D * _D) + _E * 2 * _D) / 1e12
    total_flops = jnp.asarray(fl + fl, dtype=jnp.float32)
    return h2[:_N], total_flops
